# Initial kernel scaffold; baseline (speedup 1.0000x reference)
import jax
import jax.numpy as jnp
from jax.experimental import pallas as pl


def kernel(x, edge_index, z_in, z_out):
    raise NotImplementedError("write your pallas kernel here")


# speedup vs baseline: 1.2166x; 1.2166x over previous
import jax
import jax.numpy as jnp
from jax import lax
from jax.experimental import pallas as pl
from jax.experimental.pallas import tpu as pltpu
from jax.experimental.pallas import tpu_sc as plsc

N_NODES = 10000
N_EDGES = 320000
NODE_DIM = 128
MAX_DEG = 512

NC = 2
NS = 16
L = 16

HSIZE = 10240
SLICE = HSIZE // NS
EPT = N_EDGES // NS

NPT = 312
TAIL = N_NODES - NC * NS * NPT
CHUNK = 104
NCHUNK = NPT // CHUNK

_mesh = plsc.VectorSubcoreMesh(core_axis_name="c", subcore_axis_name="s",
                               num_cores=NC, num_subcores=NS)


def _degree_kernel(src_hbm, dst_hbm, din_hbm, dout_hbm,
                   idx_v, hist_v, buf_v, acc_v, stage_sh, sem):
    cid = lax.axis_index("c")
    sid = lax.axis_index("s")

    zeros = jnp.zeros((L,), jnp.int32)

    def zero_body(i, c):
        hist_v[pl.ds(i * L, L)] = zeros
        return c
    lax.fori_loop(0, HSIZE // L, zero_body, 0)

    base = sid * EPT

    @pl.when(cid == 0)
    def _():
        pltpu.sync_copy(dst_hbm.at[pl.ds(base, EPT)], idx_v)

    @pl.when(cid == 1)
    def _():
        pltpu.sync_copy(src_hbm.at[pl.ds(base, EPT)], idx_v)

    ones = jnp.ones((L,), jnp.int32)

    def scat_body(i, c):
        idx = idx_v[pl.ds(i * L, L)]
        plsc.addupdate_scatter(hist_v, [idx], ones)
        return c
    lax.fori_loop(0, EPT // L, scat_body, 0)

    pltpu.sync_copy(hist_v, stage_sh.at[sid])
    plsc.subcore_barrier()

    col = sid * SLICE
    pltpu.sync_copy(stage_sh.at[0, pl.ds(col, SLICE)], acc_v)

    def red_body(t, c):
        pltpu.sync_copy(stage_sh.at[t, pl.ds(col, SLICE)], buf_v)

        def add_body(j, c2):
            s = pl.ds(j * L, L)
            acc_v[s] = acc_v[s] + buf_v[s]
            return c2
        lax.fori_loop(0, SLICE // L, add_body, 0)
        return c
    lax.fori_loop(1, NS, red_body, 0)

    cap = jnp.full((L,), MAX_DEG - 1, jnp.int32)

    def clamp_body(j, c):
        s = pl.ds(j * L, L)
        acc_v[s] = jnp.minimum(acc_v[s], cap)
        return c
    lax.fori_loop(0, SLICE // L, clamp_body, 0)

    @pl.when(cid == 0)
    def _():
        pltpu.sync_copy(acc_v, din_hbm.at[pl.ds(col, SLICE)])

    @pl.when(cid == 1)
    def _():
        pltpu.sync_copy(acc_v, dout_hbm.at[pl.ds(col, SLICE)])


def _gather_kernel(x_hbm, zin_hbm, zout_hbm, din_hbm, dout_hbm, out_hbm,
                   di_v, do_v, xacc_v, a_v, b_v, di16_v, do16_v,
                   sem_a, sem_b):
    cid = lax.axis_index("c")
    sid = lax.axis_index("s")
    wid = sid * NC + cid

    def process(nbase, nrows, nchunk, chunk, di, do):
        for j in range(nchunk):
            pltpu.sync_copy(din_hbm.at[pl.ds(nbase + j * chunk, chunk)],
                            di.at[j])
            pltpu.sync_copy(dout_hbm.at[pl.ds(nbase + j * chunk, chunk)],
                            do.at[j])
        cps = []
        for j in range(nchunk):
            r = pl.ds(j * chunk, chunk)
            cps.append(pltpu.async_copy(zin_hbm.at[di.at[j]],
                                        a_v.at[r], sem_a))
            cps.append(pltpu.async_copy(zout_hbm.at[do.at[j]],
                                        b_v.at[r], sem_b))
        pltpu.sync_copy(x_hbm.at[pl.ds(nbase, nrows)],
                        xacc_v.at[pl.ds(0, nrows)])
        for cp in cps:
            cp.wait()

        def add_body(i, c):
            for j in range(NODE_DIM // L):
                s = pl.ds(j * L, L)
                xacc_v[i, s] = xacc_v[i, s] + a_v[i, s] + b_v[i, s]
            return c
        lax.fori_loop(0, nrows, add_body, 0)
        pltpu.sync_copy(xacc_v.at[pl.ds(0, nrows)],
                        out_hbm.at[pl.ds(nbase, nrows)])

    process(wid * NPT, NPT, NCHUNK, CHUNK, di_v, do_v)

    @pl.when(wid == NC * NS - 1)
    def _():
        process(NC * NS * NPT, TAIL, 1, TAIL, di16_v, do16_v)


def kernel(x, edge_index, z_in, z_out):
    edge_index = edge_index.astype(jnp.int32)
    src = edge_index[0]
    dst = edge_index[1]

    deg_call = pl.kernel(
        _degree_kernel,
        out_type=[jax.ShapeDtypeStruct((HSIZE,), jnp.int32),
                  jax.ShapeDtypeStruct((HSIZE,), jnp.int32)],
        mesh=_mesh,
        scratch_types=[
            pltpu.VMEM((EPT,), jnp.int32),
            pltpu.VMEM((HSIZE,), jnp.int32),
            pltpu.VMEM((SLICE,), jnp.int32),
            pltpu.VMEM((SLICE,), jnp.int32),
            pltpu.VMEM_SHARED((NS, HSIZE), jnp.int32),
            pltpu.SemaphoreType.DMA,
        ],
        compiler_params=pltpu.CompilerParams(needs_layout_passes=False),
    )
    din, dout = deg_call(src, dst)

    gather_call = pl.kernel(
        _gather_kernel,
        out_type=jax.ShapeDtypeStruct((N_NODES, NODE_DIM), jnp.float32),
        mesh=_mesh,
        scratch_types=[
            pltpu.VMEM((NCHUNK, CHUNK), jnp.int32),
            pltpu.VMEM((NCHUNK, CHUNK), jnp.int32),
            pltpu.VMEM((NPT, NODE_DIM), jnp.float32),
            pltpu.VMEM((NPT, NODE_DIM), jnp.float32),
            pltpu.VMEM((NPT, NODE_DIM), jnp.float32),
            pltpu.VMEM((1, TAIL), jnp.int32),
            pltpu.VMEM((1, TAIL), jnp.int32),
            pltpu.SemaphoreType.DMA,
            pltpu.SemaphoreType.DMA,
        ],
    )
    return gather_call(x, z_in, z_out, din, dout)


# speedup vs baseline: 1.2348x; 1.0150x over previous
import jax
import jax.numpy as jnp
from jax import lax
from jax.experimental import pallas as pl
from jax.experimental.pallas import tpu as pltpu
from jax.experimental.pallas import tpu_sc as plsc

N_NODES = 10000
N_EDGES = 320000
NODE_DIM = 128
MAX_DEG = 512

NC = 2
NS = 16
L = 16

HSIZE = 10240
SLICE = HSIZE // NS
EPT = N_EDGES // NS
SC_UNROLL = 5

NPT = 312
TAIL = N_NODES - NC * NS * NPT
CHUNK = 104
NCHUNK = NPT // CHUNK
ROW_UNROLL = 4

_mesh = plsc.VectorSubcoreMesh(core_axis_name="c", subcore_axis_name="s",
                               num_cores=NC, num_subcores=NS)


def _degree_kernel(src_hbm, dst_hbm, din_hbm, dout_hbm,
                   idx_v, hist_v, buf_v, acc_v, stage_sh, sem):
    cid = lax.axis_index("c")
    sid = lax.axis_index("s")

    zeros = jnp.zeros((L,), jnp.int32)

    def zero_body(i, c):
        for u in range(8):
            hist_v[pl.ds((i * 8 + u) * L, L)] = zeros
        return c
    lax.fori_loop(0, HSIZE // (8 * L), zero_body, 0)

    base = sid * EPT

    @pl.when(cid == 0)
    def _():
        pltpu.sync_copy(dst_hbm.at[pl.ds(base, EPT)], idx_v)

    @pl.when(cid == 1)
    def _():
        pltpu.sync_copy(src_hbm.at[pl.ds(base, EPT)], idx_v)

    ones = jnp.ones((L,), jnp.int32)

    def scat_body(i, c):
        for u in range(SC_UNROLL):
            idx = idx_v[pl.ds((i * SC_UNROLL + u) * L, L)]
            plsc.addupdate_scatter(hist_v, [idx], ones)
        return c
    lax.fori_loop(0, EPT // (L * SC_UNROLL), scat_body, 0)

    pltpu.sync_copy(hist_v, stage_sh.at[sid])
    plsc.subcore_barrier()

    col = sid * SLICE
    pltpu.sync_copy(stage_sh.at[0, pl.ds(col, SLICE)], acc_v)

    def red_body(t, c):
        pltpu.sync_copy(stage_sh.at[t, pl.ds(col, SLICE)], buf_v)

        def add_body(j, c2):
            for u in range(8):
                s = pl.ds((j * 8 + u) * L, L)
                acc_v[s] = acc_v[s] + buf_v[s]
            return c2
        lax.fori_loop(0, SLICE // (8 * L), add_body, 0)
        return c
    lax.fori_loop(1, NS, red_body, 0)

    cap = jnp.full((L,), MAX_DEG - 1, jnp.int32)

    def clamp_body(j, c):
        for u in range(8):
            s = pl.ds((j * 8 + u) * L, L)
            acc_v[s] = jnp.minimum(acc_v[s], cap)
        return c
    lax.fori_loop(0, SLICE // (8 * L), clamp_body, 0)

    @pl.when(cid == 0)
    def _():
        pltpu.sync_copy(acc_v, din_hbm.at[pl.ds(col, SLICE)])

    @pl.when(cid == 1)
    def _():
        pltpu.sync_copy(acc_v, dout_hbm.at[pl.ds(col, SLICE)])


def _gather_kernel(x_hbm, zin_hbm, zout_hbm, din_hbm, dout_hbm, out_hbm,
                   di_v, do_v, xacc_v, a_v, b_v, di16_v, do16_v,
                   sem_a, sem_b, sem_x, sem_o):
    cid = lax.axis_index("c")
    sid = lax.axis_index("s")
    wid = sid * NC + cid

    def add_rows(row0, nrows):
        def add_body(i, c):
            for u in range(ROW_UNROLL):
                r = row0 + i * ROW_UNROLL + u
                for j in range(NODE_DIM // L):
                    s = pl.ds(j * L, L)
                    xacc_v[r, s] = xacc_v[r, s] + a_v[r, s] + b_v[r, s]
            return c
        lax.fori_loop(0, nrows // ROW_UNROLL, add_body, 0)

    nbase = wid * NPT
    cpx = pltpu.async_copy(x_hbm.at[pl.ds(nbase, NPT)], xacc_v, sem_x)
    for j in range(NCHUNK):
        pltpu.sync_copy(din_hbm.at[pl.ds(nbase + j * CHUNK, CHUNK)],
                        di_v.at[j])
        pltpu.sync_copy(dout_hbm.at[pl.ds(nbase + j * CHUNK, CHUNK)],
                        do_v.at[j])
    cps = []
    for j in range(NCHUNK):
        r = pl.ds(j * CHUNK, CHUNK)
        cps.append((pltpu.async_copy(zin_hbm.at[di_v.at[j]], a_v.at[r],
                                     sem_a),
                    pltpu.async_copy(zout_hbm.at[do_v.at[j]], b_v.at[r],
                                     sem_b)))
    cpx.wait()
    outs = []
    for j in range(NCHUNK):
        cps[j][0].wait()
        cps[j][1].wait()
        add_rows(j * CHUNK, CHUNK)
        r = pl.ds(j * CHUNK, CHUNK)
        outs.append(pltpu.async_copy(xacc_v.at[r],
                                     out_hbm.at[pl.ds(nbase + j * CHUNK,
                                                      CHUNK)], sem_o))

    @pl.when(wid == NC * NS - 1)
    def _():
        tbase = NC * NS * NPT
        pltpu.sync_copy(din_hbm.at[pl.ds(tbase, TAIL)], di16_v.at[0])
        pltpu.sync_copy(dout_hbm.at[pl.ds(tbase, TAIL)], do16_v.at[0])
        r = pl.ds(0, TAIL)
        pltpu.async_copy(zin_hbm.at[di16_v.at[0]], a_v.at[r], sem_a).wait()
        pltpu.async_copy(zout_hbm.at[do16_v.at[0]], b_v.at[r], sem_b).wait()
        pltpu.sync_copy(x_hbm.at[pl.ds(tbase, TAIL)], xacc_v.at[r])
        add_rows(0, TAIL)
        pltpu.sync_copy(xacc_v.at[r], out_hbm.at[pl.ds(tbase, TAIL)])

    for cp in outs:
        cp.wait()


def kernel(x, edge_index, z_in, z_out):
    edge_index = edge_index.astype(jnp.int32)
    src = edge_index[0]
    dst = edge_index[1]

    deg_call = pl.kernel(
        _degree_kernel,
        out_type=[jax.ShapeDtypeStruct((HSIZE,), jnp.int32),
                  jax.ShapeDtypeStruct((HSIZE,), jnp.int32)],
        mesh=_mesh,
        scratch_types=[
            pltpu.VMEM((EPT,), jnp.int32),
            pltpu.VMEM((HSIZE,), jnp.int32),
            pltpu.VMEM((SLICE,), jnp.int32),
            pltpu.VMEM((SLICE,), jnp.int32),
            pltpu.VMEM_SHARED((NS, HSIZE), jnp.int32),
            pltpu.SemaphoreType.DMA,
        ],
        compiler_params=pltpu.CompilerParams(needs_layout_passes=False),
    )
    din, dout = deg_call(src, dst)

    gather_call = pl.kernel(
        _gather_kernel,
        out_type=jax.ShapeDtypeStruct((N_NODES, NODE_DIM), jnp.float32),
        mesh=_mesh,
        scratch_types=[
            pltpu.VMEM((NCHUNK, CHUNK), jnp.int32),
            pltpu.VMEM((NCHUNK, CHUNK), jnp.int32),
            pltpu.VMEM((NPT, NODE_DIM), jnp.float32),
            pltpu.VMEM((NPT, NODE_DIM), jnp.float32),
            pltpu.VMEM((NPT, NODE_DIM), jnp.float32),
            pltpu.VMEM((1, TAIL), jnp.int32),
            pltpu.VMEM((1, TAIL), jnp.int32),
            pltpu.SemaphoreType.DMA,
            pltpu.SemaphoreType.DMA,
            pltpu.SemaphoreType.DMA,
            pltpu.SemaphoreType.DMA,
        ],
    )
    return gather_call(x, z_in, z_out, din, dout)


# speedup vs baseline: 1.8849x; 1.5265x over previous
import jax
import jax.numpy as jnp
from jax import lax
from jax.experimental import pallas as pl
from jax.experimental.pallas import tpu as pltpu
from jax.experimental.pallas import tpu_sc as plsc

N_NODES = 10000
N_EDGES = 320000
NODE_DIM = 128
MAX_DEG = 512

NC = 2
NS = 16
L = 16

HSIZE = 10240
SLICE = HSIZE // NS
EPT = N_EDGES // NS
SC_UNROLL = 5

NPT = 312
TAIL = N_NODES - NC * NS * NPT
CHUNK = 104
NCHUNK = NPT // CHUNK
ROW_UNROLL = 4

_mesh = plsc.VectorSubcoreMesh(core_axis_name="c", subcore_axis_name="s",
                               num_cores=NC, num_subcores=NS)


def _degree_kernel(src_hbm, dst_hbm, din_hbm, dout_hbm,
                   idx_v, hist_v, buf_v, acc_v, stage_sh, sem):
    cid = lax.axis_index("c")
    sid = lax.axis_index("s")

    zeros = jnp.zeros((L,), jnp.int32)

    def zero_body(i, c):
        for u in range(8):
            hist_v[pl.ds((i * 8 + u) * L, L)] = zeros
        return c
    lax.fori_loop(0, HSIZE // (8 * L), zero_body, 0)

    base = sid * EPT

    @pl.when(cid == 0)
    def _():
        pltpu.sync_copy(dst_hbm.at[pl.ds(base, EPT)], idx_v)

    @pl.when(cid == 1)
    def _():
        pltpu.sync_copy(src_hbm.at[pl.ds(base, EPT)], idx_v)

    ones = jnp.ones((L,), jnp.int32)

    def scat_body(i, c):
        for u in range(SC_UNROLL):
            idx = idx_v[pl.ds((i * SC_UNROLL + u) * L, L)]
            plsc.addupdate_scatter(hist_v, [idx], ones)
        return c
    lax.fori_loop(0, EPT // (L * SC_UNROLL), scat_body, 0)

    pltpu.sync_copy(hist_v, stage_sh.at[sid])
    plsc.subcore_barrier()

    col = sid * SLICE
    pltpu.sync_copy(stage_sh.at[0, pl.ds(col, SLICE)], acc_v)

    def red_body(t, c):
        pltpu.sync_copy(stage_sh.at[t, pl.ds(col, SLICE)], buf_v)

        def add_body(j, c2):
            for u in range(8):
                s = pl.ds((j * 8 + u) * L, L)
                acc_v[s] = acc_v[s] + buf_v[s]
            return c2
        lax.fori_loop(0, SLICE // (8 * L), add_body, 0)
        return c
    lax.fori_loop(1, NS, red_body, 0)

    cap = jnp.full((L,), MAX_DEG - 1, jnp.int32)

    def clamp_body(j, c):
        for u in range(8):
            s = pl.ds((j * 8 + u) * L, L)
            acc_v[s] = jnp.minimum(acc_v[s], cap)
        return c
    lax.fori_loop(0, SLICE // (8 * L), clamp_body, 0)

    @pl.when(cid == 0)
    def _():
        pltpu.sync_copy(acc_v, din_hbm.at[pl.ds(col, SLICE)])

    @pl.when(cid == 1)
    def _():
        pltpu.sync_copy(acc_v, dout_hbm.at[pl.ds(col, SLICE)])


def _gather_kernel(x_hbm, zin_hbm, zout_hbm, din_hbm, dout_hbm, out_hbm,
                   di_v, do_v, xacc_v, a_v, b_v, di16_v, do16_v,
                   zin_sh, zout_sh,
                   sem_a, sem_b, sem_x, sem_o, sem_t):
    cid = lax.axis_index("c")
    sid = lax.axis_index("s")
    wid = sid * NC + cid

    trows = MAX_DEG // NS
    tr = pl.ds(sid * trows, trows)
    ct1 = pltpu.async_copy(zin_hbm.at[tr], zin_sh.at[tr], sem_t)
    ct2 = pltpu.async_copy(zout_hbm.at[tr], zout_sh.at[tr], sem_t)

    def add_rows(row0, nrows):
        def add_body(i, c):
            for u in range(ROW_UNROLL):
                r = row0 + i * ROW_UNROLL + u
                for j in range(NODE_DIM // L):
                    s = pl.ds(j * L, L)
                    xacc_v[r, s] = xacc_v[r, s] + a_v[r, s] + b_v[r, s]
            return c
        lax.fori_loop(0, nrows // ROW_UNROLL, add_body, 0)

    nbase = wid * NPT
    cpx = pltpu.async_copy(x_hbm.at[pl.ds(nbase, NPT)], xacc_v, sem_x)
    for j in range(NCHUNK):
        pltpu.sync_copy(din_hbm.at[pl.ds(nbase + j * CHUNK, CHUNK)],
                        di_v.at[j])
        pltpu.sync_copy(dout_hbm.at[pl.ds(nbase + j * CHUNK, CHUNK)],
                        do_v.at[j])
    ct1.wait()
    ct2.wait()
    plsc.subcore_barrier()
    cps = []
    for j in range(NCHUNK):
        r = pl.ds(j * CHUNK, CHUNK)
        cps.append((pltpu.async_copy(zin_sh.at[di_v.at[j]], a_v.at[r],
                                     sem_a),
                    pltpu.async_copy(zout_sh.at[do_v.at[j]], b_v.at[r],
                                     sem_b)))
    cpx.wait()
    outs = []
    for j in range(NCHUNK):
        cps[j][0].wait()
        cps[j][1].wait()
        add_rows(j * CHUNK, CHUNK)
        r = pl.ds(j * CHUNK, CHUNK)
        outs.append(pltpu.async_copy(xacc_v.at[r],
                                     out_hbm.at[pl.ds(nbase + j * CHUNK,
                                                      CHUNK)], sem_o))

    @pl.when(wid == NC * NS - 1)
    def _():
        tbase = NC * NS * NPT
        pltpu.sync_copy(din_hbm.at[pl.ds(tbase, TAIL)], di16_v.at[0])
        pltpu.sync_copy(dout_hbm.at[pl.ds(tbase, TAIL)], do16_v.at[0])
        r = pl.ds(0, TAIL)
        pltpu.async_copy(zin_sh.at[di16_v.at[0]], a_v.at[r], sem_a).wait()
        pltpu.async_copy(zout_sh.at[do16_v.at[0]], b_v.at[r], sem_b).wait()
        pltpu.sync_copy(x_hbm.at[pl.ds(tbase, TAIL)], xacc_v.at[r])
        add_rows(0, TAIL)
        pltpu.sync_copy(xacc_v.at[r], out_hbm.at[pl.ds(tbase, TAIL)])

    for cp in outs:
        cp.wait()


def kernel(x, edge_index, z_in, z_out):
    edge_index = edge_index.astype(jnp.int32)
    src = edge_index[0]
    dst = edge_index[1]

    deg_call = pl.kernel(
        _degree_kernel,
        out_type=[jax.ShapeDtypeStruct((HSIZE,), jnp.int32),
                  jax.ShapeDtypeStruct((HSIZE,), jnp.int32)],
        mesh=_mesh,
        scratch_types=[
            pltpu.VMEM((EPT,), jnp.int32),
            pltpu.VMEM((HSIZE,), jnp.int32),
            pltpu.VMEM((SLICE,), jnp.int32),
            pltpu.VMEM((SLICE,), jnp.int32),
            pltpu.VMEM_SHARED((NS, HSIZE), jnp.int32),
            pltpu.SemaphoreType.DMA,
        ],
        compiler_params=pltpu.CompilerParams(needs_layout_passes=False),
    )
    din, dout = deg_call(src, dst)

    gather_call = pl.kernel(
        _gather_kernel,
        out_type=jax.ShapeDtypeStruct((N_NODES, NODE_DIM), jnp.float32),
        mesh=_mesh,
        scratch_types=[
            pltpu.VMEM((NCHUNK, CHUNK), jnp.int32),
            pltpu.VMEM((NCHUNK, CHUNK), jnp.int32),
            pltpu.VMEM((NPT, NODE_DIM), jnp.float32),
            pltpu.VMEM((NPT, NODE_DIM), jnp.float32),
            pltpu.VMEM((NPT, NODE_DIM), jnp.float32),
            pltpu.VMEM((1, TAIL), jnp.int32),
            pltpu.VMEM((1, TAIL), jnp.int32),
            pltpu.VMEM_SHARED((MAX_DEG, NODE_DIM), jnp.float32),
            pltpu.VMEM_SHARED((MAX_DEG, NODE_DIM), jnp.float32),
            pltpu.SemaphoreType.DMA,
            pltpu.SemaphoreType.DMA,
            pltpu.SemaphoreType.DMA,
            pltpu.SemaphoreType.DMA,
            pltpu.SemaphoreType.DMA,
        ],
    )
    return gather_call(x, z_in, z_out, din, dout)
